# baseline (speedup 1.0000x reference)
import functools

import jax
import jax.numpy as jnp
from jax import lax
from jax.experimental import pallas as pl
from jax.experimental.pallas import tpu as pltpu
from jax.experimental.pallas import tpu_sc as plsc


NUM_CODES = 1024
CODE_DIM = 256
COMMITMENT_COST = 0.25
ROWS = 2048
GCHUNK = 128


def _argmin_body(z_ref, zsq_ref, cb_ref, csq_ref, idx_ref):
    z = z_ref[...]
    z_sq = zsq_ref[...]
    k_total = cb_ref.shape[0]
    dot = jax.lax.dot_general(
        z, cb_ref[...], (((1,), (1,)), ((), ())),
        preferred_element_type=jnp.float32)
    dist = z_sq - 2 * dot + csq_ref[...]
    mval = jnp.min(dist, axis=-1, keepdims=True)
    iota_k = jax.lax.broadcasted_iota(jnp.int32, dist.shape, 1)
    idx = jnp.min(jnp.where(dist == mval, iota_k, k_total),
                  axis=-1).astype(jnp.int32)
    idx_ref[...] = idx[:, None]


def _st_body(z_ref, zq_ref, out_ref, loss_ref):
    z = z_ref[...]
    z_q = zq_ref[...]
    out_ref[...] = z + (z_q - z)
    diff = z_q - z
    part = jnp.sum(diff * diff).reshape(1, 1)

    @pl.when(pl.program_id(0) == 0)
    def _init():
        loss_ref[...] = part

    @pl.when(pl.program_id(0) != 0)
    def _acc():
        loss_ref[...] += part


def _make_sc_gather(V, D, BN):
    info = plsc.get_sparse_core_info()
    nw = info.num_cores * info.num_subcores
    b_per_w = BN // nw
    n_ch = b_per_w // GCHUNK
    mesh = plsc.VectorSubcoreMesh(core_axis_name="c", subcore_axis_name="s")

    @functools.partial(
        pl.kernel, mesh=mesh,
        out_type=jax.ShapeDtypeStruct((BN, D), jnp.float32),
        scratch_types=[
            pltpu.VMEM((GCHUNK,), jnp.int32),
            pltpu.VMEM((GCHUNK, D), jnp.float32),
            pltpu.SemaphoreType.DMA,
        ],
    )
    def gather(table_hbm, idx_hbm, out_hbm, idx_v, rows_v, sem):
        wid = lax.axis_index("s") * info.num_cores + lax.axis_index("c")
        base = wid * b_per_w
        for ci in range(n_ch):
            off = base + ci * GCHUNK
            pltpu.sync_copy(idx_hbm.at[pl.ds(off, GCHUNK)], idx_v)
            pltpu.async_copy(table_hbm.at[idx_v], rows_v, sem).wait()
            pltpu.sync_copy(rows_v, out_hbm.at[pl.ds(off, GCHUNK)])

    return gather


@functools.partial(jax.jit, static_argnames=())
def kernel(z_e, codebook):
    B, N, D = z_e.shape
    K = codebook.shape[0]
    flat = z_e.reshape(B * N, D)
    nblk = (B * N) // ROWS
    z_sq = jnp.sum(z_e ** 2, axis=-1, keepdims=True).reshape(B * N, 1)
    c_sq = jnp.sum(codebook ** 2, axis=-1).reshape(1, K)

    idx2d = pl.pallas_call(
        _argmin_body,
        grid=(nblk,),
        in_specs=[
            pl.BlockSpec((ROWS, D), lambda i: (i, 0)),
            pl.BlockSpec((ROWS, 1), lambda i: (i, 0)),
            pl.BlockSpec((K, D), lambda i: (0, 0)),
            pl.BlockSpec((1, K), lambda i: (0, 0)),
        ],
        out_specs=pl.BlockSpec((ROWS, 1), lambda i: (i, 0)),
        out_shape=jax.ShapeDtypeStruct((B * N, 1), jnp.int32),
    )(flat, z_sq, codebook, c_sq)

    z_q = _make_sc_gather(K, D, B * N)(codebook, idx2d.reshape(B * N))

    zq_st, loss_sum = pl.pallas_call(
        _st_body,
        grid=(nblk,),
        in_specs=[
            pl.BlockSpec((ROWS, D), lambda i: (i, 0)),
            pl.BlockSpec((ROWS, D), lambda i: (i, 0)),
        ],
        out_specs=[
            pl.BlockSpec((ROWS, D), lambda i: (i, 0)),
            pl.BlockSpec((1, 1), lambda i: (0, 0)),
        ],
        out_shape=[
            jax.ShapeDtypeStruct((B * N, D), jnp.float32),
            jax.ShapeDtypeStruct((1, 1), jnp.float32),
        ],
    )(flat, z_q)

    mean_loss = loss_sum[0, 0] / (B * N * D)
    vq_loss = mean_loss + COMMITMENT_COST * mean_loss
    return (zq_st.reshape(B, N, D), idx2d.reshape(B, N), vq_loss)


# speedup vs baseline: 1.5046x; 1.5046x over previous
import functools

import jax
import jax.numpy as jnp
from jax.experimental import pallas as pl


NUM_CODES = 1024
CODE_DIM = 256
COMMITMENT_COST = 0.25
ROWS = 2048
SUBS = 2


def _vq_sub(z, z_sq, cb, c_sq, k_total):
    dot = jax.lax.dot_general(
        z, cb, (((1,), (1,)), ((), ())),
        preferred_element_type=jnp.float32)
    dist = z_sq - 2 * dot + c_sq
    mval = jnp.min(dist, axis=-1, keepdims=True)
    iota_k = jax.lax.broadcasted_iota(jnp.int32, dist.shape, 1)
    best_idx = jnp.min(jnp.where(dist == mval, iota_k, k_total),
                       axis=-1, keepdims=True)
    onehot = (iota_k == best_idx).astype(jnp.float32)
    z_q = jax.lax.dot_general(
        onehot, cb, (((1,), (0,)), ((), ())),
        preferred_element_type=jnp.float32)
    diff = z_q - z
    return best_idx[:, 0].astype(jnp.int32), z + diff, jnp.sum(diff * diff)


def _vq_body(z_ref, zsq_ref, cb_ref, csq_ref, zq_ref, idx_ref, loss_ref):
    cb = cb_ref[...]
    c_sq = csq_ref[...]
    k_total = cb.shape[0]
    sub_rows = z_ref.shape[0] // SUBS

    part = None
    for s in range(SUBS):
        sl = pl.ds(s * sub_rows, sub_rows)
        idx_s, zq_s, loss_s = _vq_sub(
            z_ref[sl, :], zsq_ref[sl, :], cb, c_sq, k_total)
        zq_ref[sl, :] = zq_s
        idx_ref[sl, :] = idx_s[:, None]
        part = loss_s if part is None else part + loss_s

    part = part.reshape(1, 1)

    @pl.when(pl.program_id(0) == 0)
    def _init():
        loss_ref[...] = part

    @pl.when(pl.program_id(0) != 0)
    def _acc():
        loss_ref[...] += part


@functools.partial(jax.jit, static_argnames=())
def kernel(z_e, codebook):
    B, N, D = z_e.shape
    K = codebook.shape[0]
    flat = z_e.reshape(B * N, D)
    nblk = (B * N) // ROWS
    z_sq = jnp.sum(z_e ** 2, axis=-1, keepdims=True).reshape(B * N, 1)
    c_sq = jnp.sum(codebook ** 2, axis=-1).reshape(1, K)

    zq_st, idx, loss_sum = pl.pallas_call(
        _vq_body,
        grid=(nblk,),
        in_specs=[
            pl.BlockSpec((ROWS, D), lambda i: (i, 0)),
            pl.BlockSpec((ROWS, 1), lambda i: (i, 0)),
            pl.BlockSpec((K, D), lambda i: (0, 0)),
            pl.BlockSpec((1, K), lambda i: (0, 0)),
        ],
        out_specs=[
            pl.BlockSpec((ROWS, D), lambda i: (i, 0)),
            pl.BlockSpec((ROWS, 1), lambda i: (i, 0)),
            pl.BlockSpec((1, 1), lambda i: (0, 0)),
        ],
        out_shape=[
            jax.ShapeDtypeStruct((B * N, D), jnp.float32),
            jax.ShapeDtypeStruct((B * N, 1), jnp.int32),
            jax.ShapeDtypeStruct((1, 1), jnp.float32),
        ],
    )(flat, z_sq, codebook, c_sq)

    mean_loss = loss_sum[0, 0] / (B * N * D)
    vq_loss = mean_loss + COMMITMENT_COST * mean_loss
    return (zq_st.reshape(B, N, D), idx.reshape(B, N), vq_loss)


# speedup vs baseline: 1.5720x; 1.0448x over previous
import functools

import jax
import jax.numpy as jnp
from jax.experimental import pallas as pl


NUM_CODES = 1024
CODE_DIM = 256
COMMITMENT_COST = 0.25
ROWS = 2048
SUBS = 4


def _vq_sub(z, z_sq, cb, c_sq, k_total):
    dot = jax.lax.dot_general(
        z, cb, (((1,), (1,)), ((), ())),
        preferred_element_type=jnp.float32)
    dist = z_sq - 2 * dot + c_sq
    mval = jnp.min(dist, axis=-1, keepdims=True)
    iota_k = jax.lax.broadcasted_iota(jnp.int32, dist.shape, 1)
    best_idx = jnp.min(jnp.where(dist == mval, iota_k, k_total),
                       axis=-1, keepdims=True)
    onehot = (iota_k == best_idx).astype(jnp.float32)
    z_q = jax.lax.dot_general(
        onehot, cb, (((1,), (0,)), ((), ())),
        preferred_element_type=jnp.float32)
    diff = z_q - z
    return best_idx[:, 0].astype(jnp.int32), z + diff, jnp.sum(diff * diff)


def _vq_body(z_ref, zsq_ref, cb_ref, csq_ref, zq_ref, idx_ref, loss_ref):
    cb = cb_ref[...]
    c_sq = csq_ref[...]
    k_total = cb.shape[0]
    sub_rows = z_ref.shape[0] // SUBS

    part = None
    for s in range(SUBS):
        sl = pl.ds(s * sub_rows, sub_rows)
        idx_s, zq_s, loss_s = _vq_sub(
            z_ref[sl, :], zsq_ref[sl, :], cb, c_sq, k_total)
        zq_ref[sl, :] = zq_s
        idx_ref[sl, :] = idx_s[:, None]
        part = loss_s if part is None else part + loss_s

    part = part.reshape(1, 1)

    @pl.when(pl.program_id(0) == 0)
    def _init():
        loss_ref[...] = part

    @pl.when(pl.program_id(0) != 0)
    def _acc():
        loss_ref[...] += part


@functools.partial(jax.jit, static_argnames=())
def kernel(z_e, codebook):
    B, N, D = z_e.shape
    K = codebook.shape[0]
    flat = z_e.reshape(B * N, D)
    nblk = (B * N) // ROWS
    z_sq = jnp.sum(z_e ** 2, axis=-1, keepdims=True).reshape(B * N, 1)
    c_sq = jnp.sum(codebook ** 2, axis=-1).reshape(1, K)

    zq_st, idx, loss_sum = pl.pallas_call(
        _vq_body,
        grid=(nblk,),
        in_specs=[
            pl.BlockSpec((ROWS, D), lambda i: (i, 0)),
            pl.BlockSpec((ROWS, 1), lambda i: (i, 0)),
            pl.BlockSpec((K, D), lambda i: (0, 0)),
            pl.BlockSpec((1, K), lambda i: (0, 0)),
        ],
        out_specs=[
            pl.BlockSpec((ROWS, D), lambda i: (i, 0)),
            pl.BlockSpec((ROWS, 1), lambda i: (i, 0)),
            pl.BlockSpec((1, 1), lambda i: (0, 0)),
        ],
        out_shape=[
            jax.ShapeDtypeStruct((B * N, D), jnp.float32),
            jax.ShapeDtypeStruct((B * N, 1), jnp.int32),
            jax.ShapeDtypeStruct((1, 1), jnp.float32),
        ],
    )(flat, z_sq, codebook, c_sq)

    mean_loss = loss_sum[0, 0] / (B * N * D)
    vq_loss = mean_loss + COMMITMENT_COST * mean_loss
    return (zq_st.reshape(B, N, D), idx.reshape(B, N), vq_loss)
